# Initial kernel scaffold; baseline (speedup 1.0000x reference)
import jax
import jax.numpy as jnp
from jax.experimental import pallas as pl


def kernel(node_features, source_nodes, timestamps, n_layers):
    raise NotImplementedError("write your pallas kernel here")


# speedup vs baseline: 1.8581x; 1.8581x over previous
import functools

import jax
import jax.numpy as jnp
from jax import lax
from jax.experimental import pallas as pl
from jax.experimental.pallas import tpu as pltpu, tpu_sc as plsc

N_NODES = 100000
D_FEAT = 128
BATCH = 65536

NC = 2
NS = 16
NW = NC * NS
CHUNK = 128
ROWS_PER_W = BATCH // NW
N_CHUNKS = ROWS_PER_W // CHUNK


def _make_gather():
    mesh = plsc.VectorSubcoreMesh(core_axis_name="c", subcore_axis_name="s")

    @functools.partial(
        pl.kernel,
        mesh=mesh,
        out_type=jax.ShapeDtypeStruct((BATCH, D_FEAT), jnp.float32),
        scratch_types=[
            pltpu.VMEM((N_CHUNKS, CHUNK), jnp.int32),
            pltpu.VMEM((CHUNK, D_FEAT), jnp.float32),
            pltpu.VMEM((CHUNK, D_FEAT), jnp.float32),
            pltpu.SemaphoreType.DMA,
            pltpu.SemaphoreType.DMA,
            pltpu.SemaphoreType.DMA,
            pltpu.SemaphoreType.DMA,
        ],
    )
    def gather(table_hbm, idx_hbm, out_hbm, idx_v, rows0, rows1,
               gsem0, gsem1, osem0, osem1):
        wid = lax.axis_index("s") * NC + lax.axis_index("c")
        base = wid * ROWS_PER_W
        bufs = (rows0, rows1)
        gsems = (gsem0, gsem1)
        osems = (osem0, osem1)

        pltpu.sync_copy(idx_hbm.at[wid], idx_v)

        gcp = [None, None]
        ocp = [None, None]
        gcp[0] = pltpu.async_copy(table_hbm.at[idx_v.at[0]], bufs[0], gsems[0])
        for j in range(N_CHUNKS):
            nxt = (j + 1) % 2
            cur = j % 2
            if j + 1 < N_CHUNKS:
                if ocp[nxt] is not None:
                    ocp[nxt].wait()
                gcp[nxt] = pltpu.async_copy(
                    table_hbm.at[idx_v.at[j + 1]], bufs[nxt], gsems[nxt])
            gcp[cur].wait()
            ocp[cur] = pltpu.async_copy(
                bufs[cur], out_hbm.at[pl.ds(base + j * CHUNK, CHUNK)],
                osems[cur])
        ocp[(N_CHUNKS - 1) % 2].wait()
        ocp[(N_CHUNKS - 2) % 2].wait()

    return gather


_gather = _make_gather()


def kernel(node_features, source_nodes, timestamps, n_layers):
    idx = source_nodes.reshape(NW, N_CHUNKS, CHUNK)
    out = _gather(node_features, idx)
    return out + n_layers * 0


# speedup vs baseline: 1.8781x; 1.0107x over previous
import functools

import jax
import jax.numpy as jnp
from jax import lax
from jax.experimental import pallas as pl
from jax.experimental.pallas import tpu as pltpu, tpu_sc as plsc

N_NODES = 100000
D_FEAT = 128
BATCH = 65536

NC = 2
NS = 16
NW = NC * NS
CHUNK = 128
ROWS_PER_W = BATCH // NW
N_CHUNKS = ROWS_PER_W // CHUNK


def _make_gather():
    mesh = plsc.VectorSubcoreMesh(core_axis_name="c", subcore_axis_name="s")

    K = 4
    LEAD = 2

    @functools.partial(
        pl.kernel,
        mesh=mesh,
        out_type=jax.ShapeDtypeStruct((BATCH, D_FEAT), jnp.float32),
        scratch_types=[
            pltpu.VMEM((N_CHUNKS, CHUNK), jnp.int32),
        ] + [pltpu.VMEM((CHUNK, D_FEAT), jnp.float32)] * K
          + [pltpu.SemaphoreType.DMA] * (2 * K),
    )
    def gather(table_hbm, idx_hbm, out_hbm, idx_v, *bufs_and_sems):
        bufs = bufs_and_sems[:K]
        gsems = bufs_and_sems[K:2 * K]
        osems = bufs_and_sems[2 * K:3 * K]
        wid = lax.axis_index("s") * NC + lax.axis_index("c")
        base = wid * ROWS_PER_W

        pltpu.sync_copy(idx_hbm.at[wid], idx_v)

        gcp = [None] * K
        ocp = [None] * K
        for m in range(LEAD):
            gcp[m % K] = pltpu.async_copy(
                table_hbm.at[idx_v.at[m]], bufs[m % K], gsems[m % K])
        for j in range(N_CHUNKS):
            m = j + LEAD
            if m < N_CHUNKS:
                b = m % K
                if ocp[b] is not None:
                    ocp[b].wait()
                    ocp[b] = None
                gcp[b] = pltpu.async_copy(
                    table_hbm.at[idx_v.at[m]], bufs[b], gsems[b])
            gcp[j % K].wait()
            ocp[j % K] = pltpu.async_copy(
                bufs[j % K], out_hbm.at[pl.ds(base + j * CHUNK, CHUNK)],
                osems[j % K])
        for b in range(K):
            if ocp[b] is not None:
                ocp[b].wait()

    return gather


_gather = _make_gather()


def kernel(node_features, source_nodes, timestamps, n_layers):
    idx = source_nodes.reshape(NW, N_CHUNKS, CHUNK)
    out = _gather(node_features, idx)
    return out + n_layers * 0
